# Initial kernel scaffold; baseline (speedup 1.0000x reference)
import jax
import jax.numpy as jnp
from jax.experimental import pallas as pl


def kernel(zL, Wq_l, Wk_l, Wv_l, Wo_l, g_l, b_l, Wq_d, Wk_d, Wv_d, Wo_d, g_d, b_d, dictionary, bias_novel):
    raise NotImplementedError("write your pallas kernel here")


# speedup vs baseline: 8.6300x; 8.6300x over previous
import jax
import jax.numpy as jnp
from jax.experimental import pallas as pl
from jax.experimental.pallas import tpu as pltpu

D = 4
L = 2048
H = 768
NH = 12
HD = 64
NF = 4096
TK = 64
EPS = 1e-5
LAM = 1e-3

TQ = 512
TA = 256
TD = 256
TR = 256


def _f32(x):
    return x.astype(jnp.float32)


def _ln(x, g, b):
    m = jnp.mean(x, axis=1, keepdims=True)
    xc = x - m
    v = jnp.mean(xc * xc, axis=1, keepdims=True)
    return xc * jax.lax.rsqrt(v + EPS) * g + b


def _dot_t(a, w):
    return jax.lax.dot_general(a, w, (((1,), (1,)), ((), ())),
                               preferred_element_type=jnp.float32)


def _dot(a, w):
    return jax.lax.dot_general(a, w, (((1,), (0,)), ((), ())),
                               preferred_element_type=jnp.float32)


def _ln_qkv_kernel(x_ref, wq_ref, wk_ref, wv_ref, g_ref, b_ref,
                   q_ref, k_ref, v_ref):
    x = x_ref[0]
    xn = _ln(x, g_ref[0], b_ref[0])
    q_ref[0] = _dot_t(xn, wq_ref[...])
    k_ref[0] = _dot_t(xn, wk_ref[...])
    v_ref[0] = _dot_t(xn, wv_ref[...])


def _seq_attn_kernel(q_ref, k_ref, v_ref, o_ref):
    for h in range(NH):
        sl = slice(h * HD, (h + 1) * HD)
        qh = q_ref[0, :, sl]
        kh = k_ref[0, :, sl]
        vh = v_ref[0, :, sl]
        s = _dot_t(qh, kh) * (1.0 / 8.0)
        mx = jnp.max(s, axis=1, keepdims=True)
        p = jnp.exp(s - mx)
        den = jnp.sum(p, axis=1, keepdims=True)
        o_ref[0, :, sl] = _dot(p, vh) / den


def _depth_blk_kernel(x_ref, o_ref, wol_ref, gd_ref, bd_ref,
                      wq_ref, wk_ref, wv_ref, wod_ref, e_ref, out_ref):
    e = e_ref[...]
    gd = gd_ref[0]
    bd = bd_ref[0]
    xs = []
    ks = []
    vs = []
    qs = []
    for i in range(D):
        xi = x_ref[i] + _dot_t(o_ref[i], wol_ref[...])
        xs.append(xi)
        ln_i = _ln(xi, gd, bd)
        qs.append(_dot_t(ln_i, wq_ref[...]))
        ks.append(_dot_t(ln_i, wk_ref[...]))
        vs.append(_dot_t(ln_i, wv_ref[...]))
    for i in range(D):
        sij = [_dot_t(qs[i] * ks[j], e) * (1.0 / 8.0) for j in range(i + 1)]
        m = sij[0]
        for j in range(1, i + 1):
            m = jnp.maximum(m, sij[j])
        es = [jnp.exp(s - m) for s in sij]
        den = es[0]
        for j in range(1, i + 1):
            den = den + es[j]
        acc = _dot(es[0], e) * vs[0]
        for j in range(1, i + 1):
            acc = acc + _dot(es[j], e) * vs[j]
        oi = acc / _dot(den, e)
        out_ref[i] = xs[i] + _dot_t(oi, wod_ref[...])


def _sae_loss_kernel(xa_ref, xb_ref, dic_ref, bias_ref,
                     pred_ref, recon_ref, sparse_ref):
    p = pl.program_id(0)
    t = pl.program_id(1)

    @pl.when(jnp.logical_and(p == 0, t == 0))
    def _():
        pred_ref[...] = jnp.zeros_like(pred_ref)
        recon_ref[...] = jnp.zeros_like(recon_ref)
        sparse_ref[...] = jnp.zeros_like(sparse_ref)

    res = xb_ref[0] - xa_ref[0]
    logits = _dot(res, dic_ref[...]) + bias_ref[0]
    zd = jnp.maximum(logits, 0.0)
    zi = jax.lax.bitcast_convert_type(zd, jnp.int32)

    lo0 = jnp.zeros((TR, 1), jnp.int32)
    hi0 = jnp.full((TR, 1), jnp.int32(0x7F800000))

    def body(_, carry):
        lo, hi = carry
        mid = lo + jax.lax.div(hi - lo, 2)
        cnt = jnp.sum((zi >= mid).astype(jnp.int32), axis=1, keepdims=True)
        ge = cnt >= TK
        return jnp.where(ge, mid, lo), jnp.where(ge, hi, mid)

    lo, hi = jax.lax.fori_loop(0, 31, body, (lo0, hi0))

    z = jnp.where(zi >= lo, zd, 0.0)
    x_novel = _dot_t(z, dic_ref[...])
    dr = x_novel - res
    pred_ref[...] += jnp.sum(res * res).reshape(1, 1)
    recon_ref[...] += jnp.sum(dr * dr).reshape(1, 1)
    sparse_ref[...] += jnp.sum(z).reshape(1, 1)


def kernel(zL, Wq_l, Wk_l, Wv_l, Wo_l, g_l, b_l,
           Wq_d, Wk_d, Wv_d, Wo_d, g_d, b_d, dictionary, bias_novel):
    x0 = _f32(zL).reshape(D, L, H)
    g_l2 = g_l.reshape(1, H)
    b_l2 = b_l.reshape(1, H)
    g_d2 = g_d.reshape(1, H)
    b_d2 = b_d.reshape(1, H)
    bias2 = bias_novel.reshape(1, NF)
    e_mat = (jnp.arange(16, dtype=jnp.int32)[:, None]
             == (jnp.arange(H, dtype=jnp.int32) // HD)[None, :]
             ).astype(jnp.float32)
    full = lambda shp: pl.BlockSpec(shp, lambda *_: tuple(0 for _ in shp))

    q, k, v = pl.pallas_call(
        _ln_qkv_kernel,
        grid=(D, L // TQ),
        in_specs=[
            pl.BlockSpec((1, TQ, H), lambda d, c: (d, c, 0)),
            full((H, H)), full((H, H)), full((H, H)),
            full((1, H)), full((1, H)),
        ],
        out_specs=[pl.BlockSpec((1, TQ, H), lambda d, c: (d, c, 0))] * 3,
        out_shape=[jax.ShapeDtypeStruct((D, L, H), jnp.float32)] * 3,
        compiler_params=pltpu.CompilerParams(
            dimension_semantics=("arbitrary", "arbitrary")),
    )(x0, Wq_l, Wk_l, Wv_l, g_l2, b_l2)

    o_seq = pl.pallas_call(
        _seq_attn_kernel,
        grid=(D, L // TA),
        in_specs=[
            pl.BlockSpec((1, TA, H), lambda d, c: (d, c, 0)),
            pl.BlockSpec((1, L, H), lambda d, c: (d, 0, 0)),
            pl.BlockSpec((1, L, H), lambda d, c: (d, 0, 0)),
        ],
        out_specs=pl.BlockSpec((1, TA, H), lambda d, c: (d, c, 0)),
        out_shape=jax.ShapeDtypeStruct((D, L, H), jnp.float32),
        compiler_params=pltpu.CompilerParams(
            dimension_semantics=("arbitrary", "arbitrary")),
    )(q, k, v)

    x2 = pl.pallas_call(
        _depth_blk_kernel,
        grid=(L // TD,),
        in_specs=[
            pl.BlockSpec((D, TD, H), lambda t: (0, t, 0)),
            pl.BlockSpec((D, TD, H), lambda t: (0, t, 0)),
            full((H, H)), full((1, H)), full((1, H)),
            full((H, H)), full((H, H)), full((H, H)), full((H, H)),
            full((16, H)),
        ],
        out_specs=pl.BlockSpec((D, TD, H), lambda t: (0, t, 0)),
        out_shape=jax.ShapeDtypeStruct((D, L, H), jnp.float32),
        compiler_params=pltpu.CompilerParams(
            dimension_semantics=("arbitrary",)),
    )(x0, o_seq, Wo_l, g_d2, b_d2, Wq_d, Wk_d, Wv_d, Wo_d, e_mat)

    pred_s, recon_s, sparse_s = pl.pallas_call(
        _sae_loss_kernel,
        grid=(D - 1, L // TR),
        in_specs=[
            pl.BlockSpec((1, TR, H), lambda p, t: (p, t, 0)),
            pl.BlockSpec((1, TR, H), lambda p, t: (p + 1, t, 0)),
            full((H, NF)),
            full((1, NF)),
        ],
        out_specs=[pl.BlockSpec((1, 1), lambda p, t: (0, 0))] * 3,
        out_shape=[jax.ShapeDtypeStruct((1, 1), jnp.float32)] * 3,
        compiler_params=pltpu.CompilerParams(
            dimension_semantics=("arbitrary", "arbitrary")),
    )(x2, x2, dictionary, bias2)

    n_el = (D - 1) * L * H
    n_z = (D - 1) * L * NF
    loss = (pred_s[0, 0] / n_el + recon_s[0, 0] / n_el
            + LAM * sparse_s[0, 0] / n_z)
    return loss
